# Initial kernel scaffold; baseline (speedup 1.0000x reference)
import jax
import jax.numpy as jnp
from jax.experimental import pallas as pl


def kernel(x, gt_list):
    raise NotImplementedError("write your pallas kernel here")


# speedup vs baseline: 2.3473x; 2.3473x over previous
import jax
import jax.numpy as jnp
from jax.experimental import pallas as pl


def _body(x_ref, gt_ref, y_ref, g_ref):
    y_ref[0, 0] = x_ref[0, 0]
    y_ref[0, 1] = x_ref[0, 3]
    y_ref[0, 2] = -x_ref[0, 2]
    y_ref[0, 3] = -x_ref[0, 1]
    y_ref[0, 4] = x_ref[0, 6]
    y_ref[0, 5] = -x_ref[0, 5]
    y_ref[0, 6] = -x_ref[0, 4]

    @pl.when(pl.program_id(0) == 0)
    def _():
        g_ref[:, 0:1] = -gt_ref[:, 1:2]
        g_ref[:, 1:2] = gt_ref[:, 0:1]
        g_ref[:, 2:3] = -gt_ref[:, 2:3]
        g_ref[:, 3:4] = -gt_ref[:, 4:5]
        g_ref[:, 4:5] = gt_ref[:, 3:4]
        g_ref[:, 5:6] = -gt_ref[:, 5:6]
        g_ref[:, 6:7] = -gt_ref[:, 7:8]
        g_ref[:, 7:8] = gt_ref[:, 6:7]
        g_ref[:, 8:9] = -gt_ref[:, 8:9]


def kernel(x, gt_list):
    B, C, T, F = x.shape
    gt_flat = gt_list.reshape(B * 100, 9)
    y, y_gt_flat = pl.pallas_call(
        _body,
        grid=(B,),
        in_specs=[
            pl.BlockSpec((1, C, T, F), lambda b: (b, 0, 0, 0)),
            pl.BlockSpec(gt_flat.shape, lambda b: (0, 0)),
        ],
        out_specs=[
            pl.BlockSpec((1, C, T, F), lambda b: (b, 0, 0, 0)),
            pl.BlockSpec(gt_flat.shape, lambda b: (0, 0)),
        ],
        out_shape=[
            jax.ShapeDtypeStruct(x.shape, x.dtype),
            jax.ShapeDtypeStruct(gt_flat.shape, gt_flat.dtype),
        ],
    )(x, gt_flat)
    return (y, y_gt_flat.reshape(gt_list.shape))
